# baseline (speedup 1.0000x reference)
import functools

import jax
import jax.numpy as jnp
from jax.experimental import pallas as pl
from jax.experimental.pallas import tpu as pltpu

_GAMMA = 2.0
_OTA_TOPK = 10
_OTA_RADIUS = 5.0
_OTA_IOU_W = 3.0
_NEG_INF = float("-inf")


def _loss_kernel(n_anchors, cls_ref, reg_ref, anc_ref, lab_ref,
                 closs_ref, rloss_ref):
    f32 = jnp.float32
    NP = cls_ref.shape[-1]
    M = lab_ref.shape[1]

    lane = jax.lax.broadcasted_iota(jnp.int32, (1, NP), 1)
    valid_anchor = lane < n_anchors

    cls = jnp.clip(cls_ref[0], 1e-7, 1.0 - 1e-7)

    pts = [anc_ref[c:c + 1, :] for c in range(3)]
    strd = [anc_ref[c + 3:c + 4, :] for c in range(3)]
    reg = [reg_ref[0, c:c + 1, :] for c in range(6)]
    ctr = [pts[c] + reg[c] * strd[c] for c in range(3)]
    sz = [jnp.exp(reg[c + 3]) * strd[c] for c in range(3)]
    pb_lo = [ctr[c] - sz[c] / 2.0 for c in range(3)]
    pb_hi = [ctr[c] + sz[c] / 2.0 for c in range(3)]

    lab = lab_ref[0]
    g_lo = [lab[:, c:c + 1] for c in range(3)]
    g_hi = [lab[:, c + 3:c + 4] for c in range(3)]
    gvalid = g_lo[0] != -1.0

    in_box = gvalid & valid_anchor
    in_ctr = gvalid & valid_anchor
    for c in range(3):
        in_box = in_box & (g_lo[c] <= pts[c]) & (pts[c] <= g_hi[c])
        gc = (g_lo[c] + g_hi[c]) / 2.0
        lb = gc - _OTA_RADIUS * strd[c]
        ub = gc + _OTA_RADIUS * strd[c]
        in_ctr = in_ctr & (lb <= pts[c]) & (pts[c] <= ub)

    fg = jnp.max((in_box | in_ctr).astype(f32), axis=0, keepdims=True) > 0.0
    cmask = in_box & in_ctr

    iw = []
    for c in range(3):
        w = jnp.minimum(g_hi[c], pb_hi[c]) - jnp.maximum(g_lo[c], pb_lo[c])
        iw.append(jnp.clip(w, 0.0, None))
    inters = iw[0] * iw[1] * iw[2]
    area_a = ((g_hi[0] - g_lo[0]) * (g_hi[1] - g_lo[1])
              * (g_hi[2] - g_lo[2]))
    area_b = ((pb_hi[0] - pb_lo[0]) * (pb_hi[1] - pb_lo[1])
              * (pb_hi[2] - pb_lo[2]))
    union = jnp.clip(area_a + area_b - inters, 1e-8, None)
    iou = inters / union

    obj = -jnp.log(cls)
    base = -obj + _OTA_IOU_W * jnp.log(iou + 1e-8)

    n_fg = jnp.sum(fg.astype(jnp.int32), keepdims=True)

    lane_m = jax.lax.broadcasted_iota(jnp.int32, (M, NP), 1)
    work = jnp.where(fg, iou, 0.0)
    tsum = jnp.zeros((M, 1), f32)
    k0 = min(_OTA_TOPK, n_anchors)
    for _ in range(k0):
        mx = jnp.max(work, axis=1, keepdims=True)
        tsum = tsum + mx
        pos = jnp.min(jnp.where(work == mx, lane_m, NP), axis=1, keepdims=True)
        work = jnp.where(lane_m == pos, -1.0, work)
    dyn_ks = jnp.clip(tsum.astype(jnp.int32), 1, n_fg)

    active = jnp.broadcast_to(fg, (M, NP))
    matched = jnp.zeros((M, NP), jnp.bool_)
    for t in range(k0):
        sel0 = active & cmask
        has0 = jnp.max(sel0.astype(f32), axis=1, keepdims=True) > 0.0
        pool = sel0 | (active & jnp.logical_not(has0))
        bmax = jnp.max(jnp.where(pool, base, _NEG_INF), axis=1, keepdims=True)
        hit = pool & (base == bmax)
        pos = jnp.min(jnp.where(hit, lane_m, NP), axis=1, keepdims=True)
        onehot = lane_m == pos
        matched = matched | (onehot & (t < dyn_ks))
        active = active & jnp.logical_not(onehot)
    matched = matched & gvalid

    amg = jnp.sum(matched.astype(f32), axis=0, keepdims=True)
    base_v = jnp.where(gvalid, base, _NEG_INF)
    cand = jnp.where(cmask, base_v, _NEG_INF)
    have = jnp.max(cmask.astype(f32), axis=0, keepdims=True) > 0.0
    col = jnp.where(have, cand, base_v)
    colmax = jnp.max(col, axis=0, keepdims=True)
    hitg = col == colmax
    gidx = jax.lax.broadcasted_iota(jnp.int32, (M, NP), 0)
    ming = jnp.min(jnp.where(hitg, gidx, M), axis=0, keepdims=True)
    onehot_best = gidx == ming
    multi = amg > 1.0
    matchf = ((onehot_best & multi)
              | (matched & jnp.logical_not(multi))).astype(f32)

    targets = jnp.max(matchf, axis=0, keepdims=True)
    K = jnp.sum(matchf, keepdims=True)

    eps = 1e-7
    iw2 = []
    for c in range(3):
        lo = jnp.maximum(pb_lo[c], g_lo[c])
        hi = jnp.minimum(pb_hi[c], g_hi[c])
        iw2.append(jnp.clip(hi - lo, 0.0, None))
    inters2 = iw2[0] * iw2[1] * iw2[2]
    union2 = area_b + area_a - inters2
    iou2 = inters2 / (union2 + eps)
    inter_diag = jnp.zeros((M, NP), f32)
    outer_diag = jnp.zeros((M, NP), f32)
    for c in range(3):
        cp = (pb_hi[c] + pb_lo[c]) / 2.0
        cb = (g_hi[c] + g_lo[c]) / 2.0
        inter_diag = inter_diag + (cb - cp) ** 2
        o1 = jnp.minimum(pb_lo[c], g_lo[c])
        o2 = jnp.maximum(pb_hi[c], g_hi[c])
        outer_diag = outer_diag + (o2 - o1) ** 2
    diou = iou2 - inter_diag / (outer_diag + eps)
    diou = jnp.clip(diou, -1.0, 1.0)
    pair_loss = 1.0 - diou
    rsum = jnp.sum(pair_loss * matchf, keepdims=True)
    rloss = jnp.where(K > 0.0, rsum / jnp.maximum(K, 1.0), 0.0)

    cmax = jnp.max(jnp.where(valid_anchor, cls, _NEG_INF), keepdims=True)
    e = jnp.where(valid_anchor, jnp.exp(cls - cmax), 0.0)
    z = jnp.sum(e, keepdims=True)
    p = jnp.sum(e * targets, keepdims=True) / z
    closs = -((1.0 - p) ** _GAMMA) * jnp.log(p + 1e-24)

    closs_ref[...] = jnp.broadcast_to(closs[None], (1, 1, 128))
    rloss_ref[...] = jnp.broadcast_to(rloss[None], (1, 1, 128))


def _run(cls_p, reg_t, anc_t, labels, n):
    B = cls_p.shape[0]
    NP = cls_p.shape[-1]
    M = labels.shape[1]
    out_shape = [jax.ShapeDtypeStruct((B, 1, 128), jnp.float32)] * 2
    in_specs = [
        pl.BlockSpec((1, 1, NP), lambda j: (j, 0, 0)),
        pl.BlockSpec((1, 6, NP), lambda j: (j, 0, 0)),
        pl.BlockSpec((6, NP), lambda j: (0, 0)),
        pl.BlockSpec((1, M, 6), lambda j: (j, 0, 0)),
    ]
    out_specs = [pl.BlockSpec((1, 1, 128), lambda j: (j, 0, 0))] * 2
    return pl.pallas_call(
        functools.partial(_loss_kernel, n),
        grid=(B,),
        in_specs=in_specs,
        out_specs=out_specs,
        out_shape=out_shape,
        compiler_params=pltpu.CompilerParams(
            dimension_semantics=("parallel",)),
    )(cls_p, reg_t, anc_t, labels)


def kernel(classifications, regressions, anchors, labels):
    B, N = classifications.shape
    NP = ((N + 1023) // 1024) * 1024
    pad = NP - N
    cls_p = jnp.pad(classifications, ((0, 0), (0, pad)),
                    constant_values=0.5)[:, None, :]
    reg_t = jnp.pad(regressions,
                    ((0, 0), (0, pad), (0, 0))).transpose(0, 2, 1)
    anc_t = jnp.pad(anchors, ((0, pad), (0, 0))).T
    closs, rloss = _run(cls_p, reg_t, anc_t, labels, N)
    return closs[:, 0, 0].mean(), rloss[:, 0, 0].mean()


# speedup vs baseline: 1.0316x; 1.0316x over previous
import functools

import jax
import jax.numpy as jnp
from jax.experimental import pallas as pl
from jax.experimental.pallas import tpu as pltpu

_GAMMA = 2.0
_OTA_TOPK = 10
_OTA_RADIUS = 5.0
_OTA_IOU_W = 3.0
_NEG_INF = float("-inf")


def _loss_kernel(n_anchors, cls_ref, reg_ref, anc_ref, lab_ref,
                 closs_ref, rloss_ref):
    f32 = jnp.float32
    NP = cls_ref.shape[-1]
    M = lab_ref.shape[1]

    lane = jax.lax.broadcasted_iota(jnp.int32, (1, NP), 1)
    valid_anchor = lane < n_anchors

    cls = jnp.clip(cls_ref[0], 1e-7, 1.0 - 1e-7)

    pts = [anc_ref[c:c + 1, :] for c in range(3)]
    strd = [anc_ref[c + 3:c + 4, :] for c in range(3)]
    reg = [reg_ref[0, c:c + 1, :] for c in range(6)]
    ctr = [pts[c] + reg[c] * strd[c] for c in range(3)]
    sz = [jnp.exp(reg[c + 3]) * strd[c] for c in range(3)]
    pb_lo = [ctr[c] - sz[c] / 2.0 for c in range(3)]
    pb_hi = [ctr[c] + sz[c] / 2.0 for c in range(3)]

    lab = lab_ref[0]
    g_lo = [lab[:, c:c + 1] for c in range(3)]
    g_hi = [lab[:, c + 3:c + 4] for c in range(3)]
    gvalid = g_lo[0] != -1.0

    d_box = None
    d_ctr = None
    for c in range(3):
        m = jnp.minimum(pts[c] - g_lo[c], g_hi[c] - pts[c])
        gc = (g_lo[c] + g_hi[c]) / 2.0
        lb = gc - _OTA_RADIUS * strd[c]
        ub = gc + _OTA_RADIUS * strd[c]
        n_ = jnp.minimum(pts[c] - lb, ub - pts[c])
        d_box = m if d_box is None else jnp.minimum(d_box, m)
        d_ctr = n_ if d_ctr is None else jnp.minimum(d_ctr, n_)

    gv_and_anchor = gvalid & valid_anchor
    d_any = jnp.where(gv_and_anchor, jnp.maximum(d_box, d_ctr), -1.0)
    fg = jnp.max(d_any, axis=0, keepdims=True) >= 0.0
    cmask = (jnp.minimum(d_box, d_ctr) >= 0.0) & gv_and_anchor

    iw = []
    for c in range(3):
        w = jnp.minimum(g_hi[c], pb_hi[c]) - jnp.maximum(g_lo[c], pb_lo[c])
        iw.append(jnp.clip(w, 0.0, None))
    inters = iw[0] * iw[1] * iw[2]
    area_a = ((g_hi[0] - g_lo[0]) * (g_hi[1] - g_lo[1])
              * (g_hi[2] - g_lo[2]))
    area_b = ((pb_hi[0] - pb_lo[0]) * (pb_hi[1] - pb_lo[1])
              * (pb_hi[2] - pb_lo[2]))
    union = jnp.clip(area_a + area_b - inters, 1e-8, None)
    iou = inters / union

    obj = -jnp.log(cls)
    base = -obj + _OTA_IOU_W * jnp.log(iou + 1e-8)

    n_fg = jnp.sum(fg.astype(jnp.int32), keepdims=True)

    lane_m = jax.lax.broadcasted_iota(jnp.int32, (M, NP), 1)
    work = jnp.where(fg, iou, 0.0)
    tsum = jnp.zeros((M, 1), f32)
    k0 = min(_OTA_TOPK, n_anchors)
    for _ in range(k0):
        mx = jnp.max(work, axis=1, keepdims=True)
        tsum = tsum + mx
        pos = jnp.min(jnp.where(work == mx, lane_m, NP), axis=1, keepdims=True)
        work = jnp.where(lane_m == pos, -1.0, work)
    dyn_ks = jnp.clip(tsum.astype(jnp.int32), 1, n_fg)

    active = jnp.broadcast_to(fg, (M, NP))
    matched = jnp.zeros((M, NP), jnp.bool_)
    for t in range(k0):
        sel0 = active & cmask
        has0 = jnp.max(sel0.astype(f32), axis=1, keepdims=True) > 0.0
        pool = sel0 | (active & jnp.logical_not(has0))
        bmax = jnp.max(jnp.where(pool, base, _NEG_INF), axis=1, keepdims=True)
        hit = pool & (base == bmax)
        pos = jnp.min(jnp.where(hit, lane_m, NP), axis=1, keepdims=True)
        onehot = lane_m == pos
        matched = matched | (onehot & (t < dyn_ks))
        active = active & jnp.logical_not(onehot)
    matched = matched & gvalid

    amg = jnp.sum(matched.astype(f32), axis=0, keepdims=True)
    base_v = jnp.where(gvalid, base, _NEG_INF)
    cand = jnp.where(cmask, base_v, _NEG_INF)
    have = jnp.max(cmask.astype(f32), axis=0, keepdims=True) > 0.0
    col = jnp.where(have, cand, base_v)
    colmax = jnp.max(col, axis=0, keepdims=True)
    hitg = col == colmax
    gidx = jax.lax.broadcasted_iota(jnp.int32, (M, NP), 0)
    ming = jnp.min(jnp.where(hitg, gidx, M), axis=0, keepdims=True)
    onehot_best = gidx == ming
    multi = amg > 1.0
    matchf = ((onehot_best & multi)
              | (matched & jnp.logical_not(multi))).astype(f32)

    targets = jnp.max(matchf, axis=0, keepdims=True)
    K = jnp.sum(matchf, keepdims=True)

    eps = 1e-7
    iw2 = []
    for c in range(3):
        lo = jnp.maximum(pb_lo[c], g_lo[c])
        hi = jnp.minimum(pb_hi[c], g_hi[c])
        iw2.append(jnp.clip(hi - lo, 0.0, None))
    inters2 = iw2[0] * iw2[1] * iw2[2]
    union2 = area_b + area_a - inters2
    iou2 = inters2 / (union2 + eps)
    inter_diag = jnp.zeros((M, NP), f32)
    outer_diag = jnp.zeros((M, NP), f32)
    for c in range(3):
        cp = (pb_hi[c] + pb_lo[c]) / 2.0
        cb = (g_hi[c] + g_lo[c]) / 2.0
        inter_diag = inter_diag + (cb - cp) ** 2
        o1 = jnp.minimum(pb_lo[c], g_lo[c])
        o2 = jnp.maximum(pb_hi[c], g_hi[c])
        outer_diag = outer_diag + (o2 - o1) ** 2
    diou = iou2 - inter_diag / (outer_diag + eps)
    diou = jnp.clip(diou, -1.0, 1.0)
    pair_loss = 1.0 - diou
    rsum = jnp.sum(pair_loss * matchf, keepdims=True)
    rloss = jnp.where(K > 0.0, rsum / jnp.maximum(K, 1.0), 0.0)

    cmax = jnp.max(jnp.where(valid_anchor, cls, _NEG_INF), keepdims=True)
    e = jnp.where(valid_anchor, jnp.exp(cls - cmax), 0.0)
    z = jnp.sum(e, keepdims=True)
    p = jnp.sum(e * targets, keepdims=True) / z
    closs = -((1.0 - p) ** _GAMMA) * jnp.log(p + 1e-24)

    closs_ref[...] = jnp.broadcast_to(closs[None], (1, 1, 128))
    rloss_ref[...] = jnp.broadcast_to(rloss[None], (1, 1, 128))


def _run(cls_p, reg_t, anc_t, labels, n):
    B = cls_p.shape[0]
    NP = cls_p.shape[-1]
    M = labels.shape[1]
    out_shape = [jax.ShapeDtypeStruct((B, 1, 128), jnp.float32)] * 2
    in_specs = [
        pl.BlockSpec((1, 1, NP), lambda j: (j, 0, 0)),
        pl.BlockSpec((1, 6, NP), lambda j: (j, 0, 0)),
        pl.BlockSpec((6, NP), lambda j: (0, 0)),
        pl.BlockSpec((1, M, 6), lambda j: (j, 0, 0)),
    ]
    out_specs = [pl.BlockSpec((1, 1, 128), lambda j: (j, 0, 0))] * 2
    return pl.pallas_call(
        functools.partial(_loss_kernel, n),
        grid=(B,),
        in_specs=in_specs,
        out_specs=out_specs,
        out_shape=out_shape,
        compiler_params=pltpu.CompilerParams(
            dimension_semantics=("parallel",)),
    )(cls_p, reg_t, anc_t, labels)


def kernel(classifications, regressions, anchors, labels):
    B, N = classifications.shape
    NP = ((N + 1023) // 1024) * 1024
    pad = NP - N
    cls_p = jnp.pad(classifications, ((0, 0), (0, pad)),
                    constant_values=0.5)[:, None, :]
    reg_t = jnp.pad(regressions,
                    ((0, 0), (0, pad), (0, 0))).transpose(0, 2, 1)
    anc_t = jnp.pad(anchors, ((0, pad), (0, 0))).T
    closs, rloss = _run(cls_p, reg_t, anc_t, labels, N)
    return closs[:, 0, 0].mean(), rloss[:, 0, 0].mean()


# speedup vs baseline: 1.2369x; 1.1990x over previous
import functools

import jax
import jax.numpy as jnp
from jax.experimental import pallas as pl
from jax.experimental.pallas import tpu as pltpu

_GAMMA = 2.0
_OTA_TOPK = 10
_OTA_RADIUS = 5.0
_OTA_IOU_W = 3.0
_NEG_INF = float("-inf")


def _loss_kernel(n_anchors, cls_ref, reg_ref, anc_ref, lab_ref,
                 closs_ref, rloss_ref):
    f32 = jnp.float32
    NP = cls_ref.shape[-1]
    M = lab_ref.shape[1]

    lane = jax.lax.broadcasted_iota(jnp.int32, (1, NP), 1)
    valid_anchor = lane < n_anchors

    cls = jnp.clip(cls_ref[0], 1e-7, 1.0 - 1e-7)

    pts = [anc_ref[c:c + 1, :] for c in range(3)]
    strd = [anc_ref[c + 3:c + 4, :] for c in range(3)]
    reg = [reg_ref[0, c:c + 1, :] for c in range(6)]
    ctr = [pts[c] + reg[c] * strd[c] for c in range(3)]
    sz = [jnp.exp(reg[c + 3]) * strd[c] for c in range(3)]
    pb_lo = [ctr[c] - sz[c] / 2.0 for c in range(3)]
    pb_hi = [ctr[c] + sz[c] / 2.0 for c in range(3)]

    lab = lab_ref[0]
    g_lo = [lab[:, c:c + 1] for c in range(3)]
    g_hi = [lab[:, c + 3:c + 4] for c in range(3)]
    gvalid = g_lo[0] != -1.0

    d_box = None
    d_ctr = None
    for c in range(3):
        m = jnp.minimum(pts[c] - g_lo[c], g_hi[c] - pts[c])
        gc = (g_lo[c] + g_hi[c]) / 2.0
        lb = gc - _OTA_RADIUS * strd[c]
        ub = gc + _OTA_RADIUS * strd[c]
        n_ = jnp.minimum(pts[c] - lb, ub - pts[c])
        d_box = m if d_box is None else jnp.minimum(d_box, m)
        d_ctr = n_ if d_ctr is None else jnp.minimum(d_ctr, n_)

    gv_and_anchor = gvalid & valid_anchor
    d_any = jnp.where(gv_and_anchor, jnp.maximum(d_box, d_ctr), -1.0)
    fg = jnp.max(d_any, axis=0, keepdims=True) >= 0.0
    cmask = (jnp.minimum(d_box, d_ctr) >= 0.0) & gv_and_anchor

    iw = []
    for c in range(3):
        w = jnp.minimum(g_hi[c], pb_hi[c]) - jnp.maximum(g_lo[c], pb_lo[c])
        iw.append(jnp.clip(w, 0.0, None))
    inters = iw[0] * iw[1] * iw[2]
    area_a = ((g_hi[0] - g_lo[0]) * (g_hi[1] - g_lo[1])
              * (g_hi[2] - g_lo[2]))
    area_b = ((pb_hi[0] - pb_lo[0]) * (pb_hi[1] - pb_lo[1])
              * (pb_hi[2] - pb_lo[2]))
    union = jnp.clip(area_a + area_b - inters, 1e-8, None)
    iou = inters / union

    obj = -jnp.log(cls)
    base = -obj + _OTA_IOU_W * jnp.log(iou + 1e-8)

    n_fg = jnp.sum(fg.astype(jnp.int32), keepdims=True)

    lane_m = jax.lax.broadcasted_iota(jnp.int32, (M, NP), 1)
    work = jnp.where(fg, iou, 0.0)
    tsum = jnp.zeros((M, 1), f32)
    k0 = min(_OTA_TOPK, n_anchors)
    for _ in range(k0):
        mx = jnp.max(work, axis=1, keepdims=True)
        tsum = tsum + mx
        pos = jnp.min(jnp.where(work == mx, lane_m, NP), axis=1, keepdims=True)
        work = jnp.where(lane_m == pos, -1.0, work)
    dyn_ks = jnp.clip(tsum.astype(jnp.int32), 1, n_fg)

    fgm = jnp.broadcast_to(fg, (M, NP))
    b0 = jnp.where(cmask, base, _NEG_INF)
    b1 = jnp.where(fgm & jnp.logical_not(cmask), base, _NEG_INF)
    rank = jnp.full((M, NP), f32(k0), f32)
    for t in range(k0):
        m0 = jnp.max(b0, axis=1, keepdims=True)
        m1 = jnp.max(b1, axis=1, keepdims=True)
        has0 = m0 > _NEG_INF
        bsel = jnp.where(has0, m0, m1)
        alive = bsel > _NEG_INF
        hit = (((b0 == bsel) & has0)
               | ((b1 == bsel) & jnp.logical_not(has0)))
        pos = jnp.min(jnp.where(hit, lane_m, NP), axis=1, keepdims=True)
        onehot = (lane_m == pos) & alive
        rank = jnp.where(onehot, f32(t), rank)
        b0 = jnp.where(onehot, _NEG_INF, b0)
        b1 = jnp.where(onehot, _NEG_INF, b1)
    matched = (rank < dyn_ks.astype(f32)) & gvalid

    amg = jnp.sum(matched.astype(f32), axis=0, keepdims=True)
    base_v = jnp.where(gvalid, base, _NEG_INF)
    cand = jnp.where(cmask, base_v, _NEG_INF)
    have = jnp.max(cmask.astype(f32), axis=0, keepdims=True) > 0.0
    col = jnp.where(have, cand, base_v)
    colmax = jnp.max(col, axis=0, keepdims=True)
    hitg = col == colmax
    gidx = jax.lax.broadcasted_iota(jnp.int32, (M, NP), 0)
    ming = jnp.min(jnp.where(hitg, gidx, M), axis=0, keepdims=True)
    onehot_best = gidx == ming
    multi = amg > 1.0
    matchf = ((onehot_best & multi)
              | (matched & jnp.logical_not(multi))).astype(f32)

    targets = jnp.max(matchf, axis=0, keepdims=True)
    K = jnp.sum(matchf, keepdims=True)

    eps = 1e-7
    iw2 = []
    for c in range(3):
        lo = jnp.maximum(pb_lo[c], g_lo[c])
        hi = jnp.minimum(pb_hi[c], g_hi[c])
        iw2.append(jnp.clip(hi - lo, 0.0, None))
    inters2 = iw2[0] * iw2[1] * iw2[2]
    union2 = area_b + area_a - inters2
    iou2 = inters2 / (union2 + eps)
    inter_diag = jnp.zeros((M, NP), f32)
    outer_diag = jnp.zeros((M, NP), f32)
    for c in range(3):
        cp = (pb_hi[c] + pb_lo[c]) / 2.0
        cb = (g_hi[c] + g_lo[c]) / 2.0
        inter_diag = inter_diag + (cb - cp) ** 2
        o1 = jnp.minimum(pb_lo[c], g_lo[c])
        o2 = jnp.maximum(pb_hi[c], g_hi[c])
        outer_diag = outer_diag + (o2 - o1) ** 2
    diou = iou2 - inter_diag / (outer_diag + eps)
    diou = jnp.clip(diou, -1.0, 1.0)
    pair_loss = 1.0 - diou
    rsum = jnp.sum(pair_loss * matchf, keepdims=True)
    rloss = jnp.where(K > 0.0, rsum / jnp.maximum(K, 1.0), 0.0)

    cmax = jnp.max(jnp.where(valid_anchor, cls, _NEG_INF), keepdims=True)
    e = jnp.where(valid_anchor, jnp.exp(cls - cmax), 0.0)
    z = jnp.sum(e, keepdims=True)
    p = jnp.sum(e * targets, keepdims=True) / z
    closs = -((1.0 - p) ** _GAMMA) * jnp.log(p + 1e-24)

    closs_ref[...] = jnp.broadcast_to(closs[None], (1, 1, 128))
    rloss_ref[...] = jnp.broadcast_to(rloss[None], (1, 1, 128))


def _run(cls_p, reg_t, anc_t, labels, n):
    B = cls_p.shape[0]
    NP = cls_p.shape[-1]
    M = labels.shape[1]
    out_shape = [jax.ShapeDtypeStruct((B, 1, 128), jnp.float32)] * 2
    in_specs = [
        pl.BlockSpec((1, 1, NP), lambda j: (j, 0, 0)),
        pl.BlockSpec((1, 6, NP), lambda j: (j, 0, 0)),
        pl.BlockSpec((6, NP), lambda j: (0, 0)),
        pl.BlockSpec((1, M, 6), lambda j: (j, 0, 0)),
    ]
    out_specs = [pl.BlockSpec((1, 1, 128), lambda j: (j, 0, 0))] * 2
    return pl.pallas_call(
        functools.partial(_loss_kernel, n),
        grid=(B,),
        in_specs=in_specs,
        out_specs=out_specs,
        out_shape=out_shape,
        compiler_params=pltpu.CompilerParams(
            dimension_semantics=("parallel",)),
    )(cls_p, reg_t, anc_t, labels)


def kernel(classifications, regressions, anchors, labels):
    B, N = classifications.shape
    NP = ((N + 1023) // 1024) * 1024
    pad = NP - N
    cls_p = jnp.pad(classifications, ((0, 0), (0, pad)),
                    constant_values=0.5)[:, None, :]
    reg_t = jnp.pad(regressions,
                    ((0, 0), (0, pad), (0, 0))).transpose(0, 2, 1)
    anc_t = jnp.pad(anchors, ((0, pad), (0, 0))).T
    closs, rloss = _run(cls_p, reg_t, anc_t, labels, N)
    return closs[:, 0, 0].mean(), rloss[:, 0, 0].mean()
